# Initial kernel scaffold; baseline (speedup 1.0000x reference)
import jax
import jax.numpy as jnp
from jax.experimental import pallas as pl


def kernel(x, edge_index_r0, edge_index_r1, W1_r0, b1_r0, W1_r1, b1_r1, W2_r0, b2_r0, W2_r1, b2_r1):
    raise NotImplementedError("write your pallas kernel here")


# speedup vs baseline: 4.6203x; 4.6203x over previous
import functools

import jax
import jax.numpy as jnp
from jax import lax
from jax.experimental import pallas as pl
from jax.experimental.pallas import tpu as pltpu
from jax.experimental.pallas import tpu_sc as plsc

N = 10000
D = 128
E = 320000
NC = 2
NS = 16
CHUNK = 128
NP = 10112
EP = 327680
SS_CH = 160
IB = 32
CH2 = 64
SS_CH2 = 320
SSB = 64
DG_CH = 320
RPS = NP // NS


def _zero_fill(ref, rows, cols):
    zero16 = jnp.zeros((16,), jnp.float32)

    def body(r, _):
        for c in range(cols // 16):
            ref[r, pl.ds(c * 16, 16)] = zero16
        return 0

    lax.fori_loop(0, rows, body, 0)


NPR = NP // CHUNK


def _deg_body(didx, out, idx_v, acc, red, idbuf, dsrc, ddst):
    c = lax.axis_index("c")
    s = lax.axis_index("s")
    a = s // 8
    t = s % 8

    iota = lax.iota(jnp.int32, 16)
    rowsel = lax.rem(iota, 8)
    lo_mask = iota < 8
    hi_mask = iota >= 8
    one16 = jnp.ones((16,), jnp.float32)
    zero16 = jnp.zeros((16,), jnp.float32)

    for k in range(4):
        idbuf[0, pl.ds(k * 16, 16)] = iota + k * 16
    idbuf[0, pl.ds(NPR - 16, 16)] = iota + (NPR - 16)

    def zacc(i, _):
        for k in range(8):
            acc[k, pl.ds(i * 16, 16)] = zero16
        return 0

    lax.fori_loop(0, NP // 16, zacc, 0)
    _zero_fill(red, NPR, CHUNK)

    @pl.when(s == 0)
    def _():
        pltpu.sync_copy(red, dsrc)

    @pl.when(s == 1)
    def _():
        pltpu.sync_copy(red, ddst)

    plsc.subcore_barrier()

    for bi in range(DG_CH // IB):
        pltpu.sync_copy(didx.at[c, a, t, pl.ds(bi * IB, IB)], idx_v)

        def srow(j, _):
            for k in range(8):
                v = idx_v[j, pl.ds(k * 16, 16)]
                plsc.addupdate_scatter(acc, [rowsel, v], one16, mask=lo_mask)
                plsc.addupdate_scatter(acc, [rowsel, v], one16, mask=hi_mask)
            return 0

        lax.fori_loop(0, IB, srow, 0)

    def rrow(i, _):
        tot = acc[0, pl.ds(i * 16, 16)]
        for k in range(1, 8):
            tot = tot + acc[k, pl.ds(i * 16, 16)]
        red[i // 8, pl.ds((i % 8) * 16, 16)] = tot
        return 0

    lax.fori_loop(0, NP // 16, rrow, 0)

    @pl.when(a == 0)
    def _():
        pltpu.sync_copy(red, dsrc.at[idbuf.at[0]], add=True)

    @pl.when(a == 1)
    def _():
        pltpu.sync_copy(red, ddst.at[idbuf.at[0]], add=True)

    plsc.subcore_barrier()

    @pl.when(s == 0)
    def _():
        pltpu.sync_copy(dsrc, out.at[c, 0])

    @pl.when(s == 1)
    def _():
        pltpu.sync_copy(ddst, out.at[c, 1])


def _segsum_body(h0, h1, srcs, dsts, out, src_v, dst_v, r0, r1, r2, r3, acc,
                 g0, g1, g2, g3, s0, s1, s2, s3):
    c = lax.axis_index("c")
    s = lax.axis_index("s")
    rows = [r0, r1, r2, r3]
    gsem = [g0, g1, g2, g3]
    ssem = [s0, s1, s2, s3]

    _zero_fill(r0, CH2, D)
    for k in range(9):
        pltpu.sync_copy(r0, acc.at[pl.ds(s * RPS + k * CH2, CH2)])
    pltpu.sync_copy(r0.at[pl.ds(0, RPS - 9 * CH2)],
                    acc.at[pl.ds(s * RPS + 9 * CH2, RPS - 9 * CH2)])
    plsc.subcore_barrier()

    def loop_over(h):
        for bi in range(SS_CH2 // SSB):
            pltpu.sync_copy(srcs.at[c, s, pl.ds(bi * SSB, SSB)], src_v)
            pltpu.sync_copy(dsts.at[c, s, pl.ds(bi * SSB, SSB)], dst_v)
            for b in range(4):
                pltpu.async_copy(h.at[src_v.at[b]], rows[b], gsem[b])

            def body(kk, _):
                for b in range(4):
                    j = 4 * kk + b
                    pltpu.make_async_copy(h.at[src_v.at[j]], rows[b],
                                          gsem[b]).wait()
                    pltpu.async_copy(rows[b], acc.at[dst_v.at[j]], ssem[b],
                                     add=True)
                for b in range(4):
                    j = 4 * kk + b
                    pltpu.make_async_copy(rows[b], acc.at[dst_v.at[j]],
                                          ssem[b]).wait()

                    @pl.when(j + 4 < SSB)
                    def _():
                        pltpu.async_copy(h.at[src_v.at[j + 4]], rows[b],
                                         gsem[b])

                return 0

            lax.fori_loop(0, SSB // 4, body, 0)

    @pl.when(c == 0)
    def _():
        loop_over(h0)

    @pl.when(c == 1)
    def _():
        loop_over(h1)

    plsc.subcore_barrier()
    for k in range(4):
        pltpu.sync_copy(acc.at[pl.ds(s * RPS + k * CHUNK, CHUNK)],
                        out.at[c, pl.ds(s * RPS + k * CHUNK, CHUNK)])
    pltpu.sync_copy(acc.at[pl.ds(s * RPS + 4 * CHUNK, RPS - 4 * CHUNK)],
                    out.at[c, pl.ds(s * RPS + 4 * CHUNK, RPS - 4 * CHUNK)])


@functools.cache
def _sc_calls():
    mesh = plsc.VectorSubcoreMesh(core_axis_name="c", subcore_axis_name="s",
                                  num_cores=NC, num_subcores=NS)
    deg_kernel = pl.kernel(
        _deg_body,
        out_type=jax.ShapeDtypeStruct((NC, 2, NPR, CHUNK), jnp.float32),
        mesh=mesh,
        scratch_types=[
            pltpu.VMEM((IB, CHUNK), jnp.int32),
            pltpu.VMEM((8, NP), jnp.float32),
            pltpu.VMEM((NPR, CHUNK), jnp.float32),
            pltpu.VMEM((1, NPR), jnp.int32),
            pltpu.VMEM_SHARED((NPR, CHUNK), jnp.float32),
            pltpu.VMEM_SHARED((NPR, CHUNK), jnp.float32),
        ],
    )
    segsum_kernel = pl.kernel(
        _segsum_body,
        out_type=jax.ShapeDtypeStruct((NC, NP, D), jnp.float32),
        mesh=mesh,
        scratch_types=[
            pltpu.VMEM((SSB, CH2), jnp.int32),
            pltpu.VMEM((SSB, CH2), jnp.int32),
            pltpu.VMEM((CH2, D), jnp.float32),
            pltpu.VMEM((CH2, D), jnp.float32),
            pltpu.VMEM((CH2, D), jnp.float32),
            pltpu.VMEM((CH2, D), jnp.float32),
            pltpu.VMEM_SHARED((NP, D), jnp.float32),
        ] + [pltpu.SemaphoreType.DMA] * 8,
    )
    return deg_kernel, segsum_kernel


BLK = 1264
GRID = NP // BLK


def _nrm(ref):
    return lax.rsqrt(jnp.maximum(ref[...], 1.0))


def _layer1_body(x_ref, ns0_ref, ns1_ref, w0_ref, w1_ref, h0_ref, h1_ref):
    x = x_ref[...]
    h0_ref[...] = jnp.dot(x * _nrm(ns0_ref), w0_ref[...],
                          preferred_element_type=jnp.float32)
    h1_ref[...] = jnp.dot(x * _nrm(ns1_ref), w1_ref[...],
                          preferred_element_type=jnp.float32)


def _layer2_body(agg_ref, nd0_ref, nd1_ref, ns0_ref, ns1_ref, b0_ref, b1_ref,
                 w0_ref, w1_ref, g0_ref, g1_ref):
    h = 0.5 * (agg_ref[0] * _nrm(nd0_ref) + b0_ref[...]
               + agg_ref[1] * _nrm(nd1_ref) + b1_ref[...])
    h = jnp.maximum(h, 0.0)
    g0_ref[...] = jnp.dot(h * _nrm(ns0_ref), w0_ref[...],
                          preferred_element_type=jnp.float32)
    g1_ref[...] = jnp.dot(h * _nrm(ns1_ref), w1_ref[...],
                          preferred_element_type=jnp.float32)


def _final_body(agg_ref, nd0_ref, nd1_ref, b0_ref, b1_ref, o_ref):
    o_ref[...] = 0.5 * (agg_ref[0] * _nrm(nd0_ref) + b0_ref[...]
                        + agg_ref[1] * _nrm(nd1_ref) + b1_ref[...])


_row_spec = pl.BlockSpec((BLK, D), lambda i: (i, 0))
_ns_spec = pl.BlockSpec((BLK, 1), lambda i: (i, 0))
_agg_spec = pl.BlockSpec((NC, BLK, D), lambda i: (0, i, 0))
_w_spec = pl.BlockSpec((D, D), lambda i: (0, 0))
_b_spec = pl.BlockSpec((1, D), lambda i: (0, 0))

_layer1_call = pl.pallas_call(
    _layer1_body,
    grid=(GRID,),
    in_specs=[_row_spec, _ns_spec, _ns_spec, _w_spec, _w_spec],
    out_specs=[_row_spec, _row_spec],
    out_shape=[jax.ShapeDtypeStruct((NP, D), jnp.float32)] * 2,
)

_layer2_call = pl.pallas_call(
    _layer2_body,
    grid=(GRID,),
    in_specs=[_agg_spec, _ns_spec, _ns_spec, _ns_spec, _ns_spec,
              _b_spec, _b_spec, _w_spec, _w_spec],
    out_specs=[_row_spec, _row_spec],
    out_shape=[jax.ShapeDtypeStruct((NP, D), jnp.float32)] * 2,
)

_final_call = pl.pallas_call(
    _final_body,
    grid=(GRID,),
    in_specs=[_agg_spec, _ns_spec, _ns_spec, _b_spec, _b_spec],
    out_specs=_row_spec,
    out_shape=jax.ShapeDtypeStruct((NP, D), jnp.float32),
)


@jax.jit
def kernel(x, edge_index_r0, edge_index_r1, W1_r0, b1_r0, W1_r1, b1_r1,
           W2_r0, b2_r0, W2_r1, b2_r1):
    pad = jnp.full((2, EP - E), N, jnp.int32)
    e0 = jnp.concatenate([edge_index_r0.astype(jnp.int32), pad], axis=1)
    e1 = jnp.concatenate([edge_index_r1.astype(jnp.int32), pad], axis=1)

    didx = jnp.stack([e0, e1]).reshape(NC, 2, 8, DG_CH, CHUNK)
    srcs = jnp.stack([e0[0], e1[0]]).reshape(NC, NS, SS_CH2, CH2)
    dsts = jnp.stack([e0[1], e1[1]]).reshape(NC, NS, SS_CH2, CH2)

    deg_kernel, segsum_kernel = _sc_calls()
    cnt = deg_kernel(didx)
    cnt = cnt.reshape(4, NP, 1)
    ns0, nd0, ns1, nd1 = cnt[0], cnt[1], cnt[2], cnt[3]

    xp = jnp.pad(x, ((0, NP - N), (0, 0)))
    b1_r0_ = b1_r0.reshape(1, D)
    b1_r1_ = b1_r1.reshape(1, D)
    b2_r0_ = b2_r0.reshape(1, D)
    b2_r1_ = b2_r1.reshape(1, D)

    h0, h1 = _layer1_call(xp, ns0, ns1, W1_r0, W1_r1)
    agg1 = segsum_kernel(h0, h1, srcs, dsts)
    g0, g1 = _layer2_call(agg1, nd0, nd1, ns0, ns1, b1_r0_, b1_r1_, W2_r0, W2_r1)
    agg2 = segsum_kernel(g0, g1, srcs, dsts)
    out = _final_call(agg2, nd0, nd1, b2_r0_, b2_r1_)
    return out[:N]
